# baseline (speedup 1.0000x reference)
import jax
import jax.numpy as jnp
from jax import lax
from jax.experimental import pallas as pl
from jax.experimental.pallas import tpu as pltpu
from jax.experimental.pallas import tpu_sc as plsc

NUM_ENTITIES = 1000000
NUM_RELATIONS = 1000
EMBED_DIM = 64
BATCH = 16384

NC = 2
NS = 16
NW = NC * NS
COLS = BATCH // NW
LANES = 16
MV = COLS // LANES
ICH = 128
NICH = COLS // ICH


def _tec_body(heads_hbm, rels_hbm, entf_hbm, relf_hbm, outt_hbm,
              hvec, rvec, rtab, ebuf, idxb, yv, sem):
    wid = lax.axis_index("c") * NS + lax.axis_index("s")
    base = wid * COLS

    pltpu.sync_copy(heads_hbm.at[pl.ds(base, COLS)], hvec)
    pltpu.sync_copy(rels_hbm.at[pl.ds(base, COLS)], rvec)
    pltpu.sync_copy(relf_hbm, rtab)

    def fire(d, slot):
        off = d * NUM_ENTITIES
        for m in range(MV):
            idxb[slot, pl.ds(m * LANES, LANES)] = (
                hvec[pl.ds(m * LANES, LANES)] + off)
        for j in range(NICH):
            pltpu.async_copy(
                entf_hbm.at[idxb.at[slot, pl.ds(j * ICH, ICH)]],
                ebuf.at[d, pl.ds(j * ICH, ICH)], sem)

    def drain(d):
        pltpu.make_async_copy(
            entf_hbm.at[pl.ds(0, COLS)], ebuf.at[d], sem).wait()

    fire(0, 0)

    def dim_step(d, acc):
        @pl.when(d + 1 < EMBED_DIM)
        def _():
            fire(d + 1, (d + 1) % 2)

        drain(d)
        roff = d * NUM_RELATIONS
        new_acc = []
        for m in range(MV):
            ridx = rvec[pl.ds(m * LANES, LANES)] + roff
            rval = plsc.load_gather(rtab, [ridx])
            v = ebuf[d, pl.ds(m * LANES, LANES)] + rval
            ebuf[d, pl.ds(m * LANES, LANES)] = v
            new_acc.append(acc[m] + v * v)
        return tuple(new_acc)

    zero = jnp.zeros((LANES,), jnp.float32)
    acc = lax.fori_loop(0, EMBED_DIM, dim_step, (zero,) * MV)

    for m in range(MV):
        t = jnp.maximum(acc[m], jnp.float32(1e-24))
        bits = lax.bitcast_convert_type(t, jnp.int32)
        y = lax.bitcast_convert_type(
            jnp.int32(0x5F3759DF) - (bits >> 1), jnp.float32)
        for _ in range(3):
            y = y * (jnp.float32(1.5) - jnp.float32(0.5) * t * y * y)
        yv[pl.ds(m * LANES, LANES)] = y

    def scale(d, _):
        for m in range(MV):
            ebuf[d, pl.ds(m * LANES, LANES)] = (
                ebuf[d, pl.ds(m * LANES, LANES)] * yv[pl.ds(m * LANES, LANES)])
        return 0

    lax.fori_loop(0, EMBED_DIM, scale, 0)

    pltpu.sync_copy(ebuf, outt_hbm.at[:, pl.ds(base, COLS)])


@jax.jit
def _run(heads, rels, entf, relf):
    mesh = plsc.VectorSubcoreMesh(
        core_axis_name="c", subcore_axis_name="s",
        num_cores=NC, num_subcores=NS)
    return pl.kernel(
        _tec_body,
        out_type=jax.ShapeDtypeStruct((EMBED_DIM, BATCH), jnp.float32),
        mesh=mesh,
        scratch_types=[
            pltpu.VMEM((COLS,), jnp.int32),
            pltpu.VMEM((COLS,), jnp.int32),
            pltpu.VMEM((EMBED_DIM * NUM_RELATIONS,), jnp.float32),
            pltpu.VMEM((EMBED_DIM, COLS), jnp.float32),
            pltpu.VMEM((2, COLS), jnp.int32),
            pltpu.VMEM((COLS,), jnp.float32),
            pltpu.SemaphoreType.DMA,
        ],
        compiler_params=pltpu.CompilerParams(needs_layout_passes=False),
    )(heads, rels, entf, relf)


def kernel(heads, relations, entity_table, relation_table):
    heads = jnp.asarray(heads, jnp.int32)
    relations = jnp.asarray(relations, jnp.int32)
    entf = entity_table.T.reshape(EMBED_DIM * NUM_ENTITIES)
    relf = relation_table.T.reshape(EMBED_DIM * NUM_RELATIONS)
    outt = _run(heads, relations, entf, relf)
    return outt.T


# speedup vs baseline: 8.9008x; 8.9008x over previous
import jax
import jax.numpy as jnp
from jax import lax
from jax.experimental import pallas as pl
from jax.experimental.pallas import tpu as pltpu
from jax.experimental.pallas import tpu_sc as plsc

NUM_ENTITIES = 1000000
NUM_RELATIONS = 1000
EMBED_DIM = 64
BATCH = 16384

NC = 2
NS = 16
LANES = 16
DPC = EMBED_DIM // NC
CPT = BATCH // NS
ICH = 128
NICH = CPT // ICH
ALIGNED = (NUM_ENTITIES // 128) * 128
NTAIL = NUM_ENTITIES - ALIGNED
HW = ALIGNED // 2
SCH = 2
SCHW = HW // SCH
COLS_B = BATCH // (NC * NS)


def _rsqrt(t):
    bits = lax.bitcast_convert_type(t, jnp.int32)
    y = lax.bitcast_convert_type(
        jnp.int32(0x5F3759DF) - (bits >> 1), jnp.float32)
    for _ in range(3):
        y = y * (jnp.float32(1.5) - jnp.float32(0.5) * t * y * y)
    return y


def _a_body(heads_hbm, rels_hbm, entt_hbm, relf_hbm, auxf_hbm, v_hbm, sq_hbm,
            hvec, hhi, ho, haux, rvec, rtab, auxt, ebuf, elo, acc,
            row_sp, semr, semg):
    c = lax.axis_index("c")
    sid = lax.axis_index("s")
    cbase = sid * CPT

    pltpu.sync_copy(heads_hbm.at[pl.ds(cbase, CPT)], ho)
    pltpu.sync_copy(rels_hbm.at[pl.ds(cbase, CPT)], rvec)
    pltpu.sync_copy(relf_hbm.at[pl.ds(c * DPC * NUM_RELATIONS,
                                      DPC * NUM_RELATIONS)], rtab)
    pltpu.sync_copy(auxf_hbm, auxt)

    for m in range(CPT // LANES):
        sl = pl.ds(m * LANES, LANES)
        h = ho[sl]
        hvec[sl] = jnp.minimum(h, jnp.int32(HW - 1))
        hhi[sl] = jnp.clip(h - jnp.int32(HW), jnp.int32(0),
                           jnp.int32(HW - 1))
        haux[sl] = jnp.maximum(h - jnp.int32(ALIGNED), jnp.int32(0))
        acc[sl] = jnp.zeros((LANES,), jnp.float32)

    def half_pass(d, src_off, idx_ref, dst_fn):
        @pl.when(sid == 0)
        def _():
            for k in range(SCH):
                pltpu.async_copy(
                    entt_hbm.at[pl.ds(d, 1),
                                pl.ds(src_off + k * SCHW, SCHW)],
                    row_sp.at[pl.ds(0, 1), pl.ds(k * SCHW, SCHW)], semr)
            for k in range(SCH):
                pltpu.make_async_copy(
                    entt_hbm.at[pl.ds(0, 1), pl.ds(0, SCHW)],
                    row_sp.at[pl.ds(0, 1), pl.ds(0, SCHW)], semr).wait()

        plsc.subcore_barrier()

        for j in range(NICH):
            sj = pl.ds(j * ICH, ICH)
            pltpu.async_copy(
                row_sp.at[0].at[idx_ref.at[sj]], dst_fn(sj), semg)
        for j in range(NICH):
            pltpu.make_async_copy(
                relf_hbm.at[pl.ds(0, ICH)], elo.at[pl.ds(0, ICH)],
                semg).wait()

        plsc.subcore_barrier()

    def dim_step(dl, _):
        d = c * DPC + dl
        half_pass(d, 0, hvec, lambda sj: elo.at[sj])
        half_pass(d, HW, hhi, lambda sj: ebuf.at[dl, sj])

        roff = dl * NUM_RELATIONS
        aoff = (c * DPC + dl) * NTAIL
        for m in range(CPT // LANES):
            sl = pl.ds(m * LANES, LANES)
            e_hi = ebuf[dl, sl]
            e_aux = plsc.load_gather(auxt, [haux[sl] + aoff])
            h = ho[sl]
            e = jnp.where(h >= jnp.int32(HW), e_hi, elo[sl])
            e = jnp.where(h >= jnp.int32(ALIGNED), e_aux, e)
            v = e + plsc.load_gather(rtab, [rvec[sl] + roff])
            ebuf[dl, sl] = v
            acc[sl] = acc[sl] + v * v
        return 0

    lax.fori_loop(0, DPC, dim_step, 0)

    pltpu.sync_copy(ebuf, v_hbm.at[pl.ds(c * DPC, DPC), pl.ds(cbase, CPT)])
    pltpu.sync_copy(acc, sq_hbm.at[c, pl.ds(cbase, CPT)])


def _b_body(v_hbm, sq_hbm, outt_hbm, s0, s1, yv, vbuf):
    wid = lax.axis_index("c") * NS + lax.axis_index("s")
    base = wid * COLS_B

    pltpu.sync_copy(sq_hbm.at[0, pl.ds(base, COLS_B)], s0)
    pltpu.sync_copy(sq_hbm.at[1, pl.ds(base, COLS_B)], s1)
    pltpu.sync_copy(v_hbm.at[:, pl.ds(base, COLS_B)], vbuf)

    for m in range(COLS_B // LANES):
        sl = pl.ds(m * LANES, LANES)
        t = jnp.maximum(s0[sl] + s1[sl], jnp.float32(1e-24))
        yv[sl] = _rsqrt(t)

    def scale(d, _):
        for m in range(COLS_B // LANES):
            sl = pl.ds(m * LANES, LANES)
            vbuf[d, sl] = vbuf[d, sl] * yv[sl]
        return 0

    lax.fori_loop(0, EMBED_DIM, scale, 0)

    pltpu.sync_copy(vbuf, outt_hbm.at[:, pl.ds(base, COLS_B)])


@jax.jit
def _run(heads, rels, entt, relf, auxf):
    mesh = plsc.VectorSubcoreMesh(
        core_axis_name="c", subcore_axis_name="s",
        num_cores=NC, num_subcores=NS)
    v_hbm, sq_hbm = pl.kernel(
        _a_body,
        out_type=(jax.ShapeDtypeStruct((EMBED_DIM, BATCH), jnp.float32),
                  jax.ShapeDtypeStruct((NC, BATCH), jnp.float32)),
        mesh=mesh,
        scratch_types=[
            pltpu.VMEM((CPT,), jnp.int32),
            pltpu.VMEM((CPT,), jnp.int32),
            pltpu.VMEM((CPT,), jnp.int32),
            pltpu.VMEM((CPT,), jnp.int32),
            pltpu.VMEM((CPT,), jnp.int32),
            pltpu.VMEM((DPC * NUM_RELATIONS,), jnp.float32),
            pltpu.VMEM((EMBED_DIM * NTAIL,), jnp.float32),
            pltpu.VMEM((DPC, CPT), jnp.float32),
            pltpu.VMEM((CPT,), jnp.float32),
            pltpu.VMEM((CPT,), jnp.float32),
            pltpu.VMEM_SHARED((1, HW), jnp.float32),
            pltpu.SemaphoreType.DMA,
            pltpu.SemaphoreType.DMA,
        ],
        compiler_params=pltpu.CompilerParams(needs_layout_passes=False),
    )(heads, rels, entt, relf, auxf)

    return pl.kernel(
        _b_body,
        out_type=jax.ShapeDtypeStruct((EMBED_DIM, BATCH), jnp.float32),
        mesh=mesh,
        scratch_types=[
            pltpu.VMEM((COLS_B,), jnp.float32),
            pltpu.VMEM((COLS_B,), jnp.float32),
            pltpu.VMEM((COLS_B,), jnp.float32),
            pltpu.VMEM((EMBED_DIM, COLS_B), jnp.float32),
        ],
        compiler_params=pltpu.CompilerParams(needs_layout_passes=False),
    )(v_hbm, sq_hbm)


def kernel(heads, relations, entity_table, relation_table):
    heads = jnp.asarray(heads, jnp.int32)
    relations = jnp.asarray(relations, jnp.int32)
    entt = entity_table.T
    relf = relation_table.T.reshape(EMBED_DIM * NUM_RELATIONS)
    auxf = entt[:, ALIGNED:].reshape(EMBED_DIM * NTAIL)
    outt = _run(heads, relations, entt, relf, auxf)
    return outt.T


# speedup vs baseline: 12.5167x; 1.4062x over previous
import jax
import jax.numpy as jnp
from jax import lax
from jax.experimental import pallas as pl
from jax.experimental.pallas import tpu as pltpu
from jax.experimental.pallas import tpu_sc as plsc

NUM_ENTITIES = 1000000
NUM_RELATIONS = 1000
EMBED_DIM = 64
BATCH = 16384

NC = 2
NS = 16
NW = NC * NS
ROWS_PER_W = BATCH // NW
CHUNK = 64
NCHUNK = ROWS_PER_W // CHUNK
LANES = 16
NVEC = EMBED_DIM // LANES
GRP = 8
NGRP = NUM_ENTITIES // GRP

_GATHER_DNUMS = lax.GatherDimensionNumbers(
    offset_dims=(), collapsed_slice_dims=(0,), start_index_map=(0,))


def _permute(x, idx):
    return lax.gather(x, idx[:, None], _GATHER_DNUMS, (1,),
                      mode=lax.GatherScatterMode.PROMISE_IN_BOUNDS)


def _tec_body(heads_hbm, rels_hbm, ent_hbm, rel_hbm, out_hbm,
              hvec, rvec, reltab, entbuf, outbuf, sem):
    wid = lax.axis_index("c") * NS + lax.axis_index("s")
    base = wid * ROWS_PER_W

    pltpu.sync_copy(heads_hbm.at[pl.ds(base, ROWS_PER_W)],
                    hvec.at[pl.ds(0, ROWS_PER_W)])
    pltpu.sync_copy(rels_hbm.at[pl.ds(base, ROWS_PER_W)],
                    rvec.at[pl.ds(0, ROWS_PER_W)])
    pltpu.sync_copy(rel_hbm, reltab)

    iota = lax.iota(jnp.int32, LANES)
    perms = [iota ^ sh for sh in (8, 4, 2, 1)]

    def chunk(c, _):
        def fire(i, _):
            h = hvec[pl.ds(c * CHUNK + i, LANES)][0]
            pltpu.async_copy(
                ent_hbm.at[pl.ds(h, 1)], entbuf.at[pl.ds(i, 1)], sem)
            return 0

        lax.fori_loop(0, CHUNK, fire, 0)
        pltpu.make_async_copy(out_hbm.at[pl.ds(0, CHUNK)], entbuf, sem).wait()

        def row(i, _):
            r = rvec[pl.ds(c * CHUNK + i, LANES)][0]
            rq = r >> 1
            rp = (r & 1) * EMBED_DIM
            vs = []
            ss = None
            for k in range(NVEC):
                v = (entbuf[i, pl.ds(k * LANES, LANES)]
                     + reltab[rq, pl.ds(rp + k * LANES, LANES)])
                vs.append(v)
                sq = v * v
                ss = sq if ss is None else ss + sq
            for p in perms:
                ss = ss + _permute(ss, p)
            t = jnp.maximum(ss, jnp.float32(1e-24))
            bits = lax.bitcast_convert_type(t, jnp.int32)
            y = lax.bitcast_convert_type(
                jnp.int32(0x5F3759DF) - (bits >> 1), jnp.float32)
            for _ in range(3):
                y = y * (jnp.float32(1.5) - jnp.float32(0.5) * t * y * y)
            for k in range(NVEC):
                outbuf[i, pl.ds(k * LANES, LANES)] = vs[k] * y
            return 0

        lax.fori_loop(0, CHUNK, row, 0)
        pltpu.sync_copy(outbuf, out_hbm.at[pl.ds(base + c * CHUNK, CHUNK)])
        return 0

    lax.fori_loop(0, NCHUNK, chunk, 0)


@jax.jit
def _run(heads, rels, ent3, rel2):
    mesh = plsc.VectorSubcoreMesh(
        core_axis_name="c", subcore_axis_name="s",
        num_cores=NC, num_subcores=NS)
    return pl.kernel(
        _tec_body,
        out_type=jax.ShapeDtypeStruct((BATCH, EMBED_DIM), jnp.float32),
        mesh=mesh,
        scratch_types=[
            pltpu.VMEM((ROWS_PER_W + LANES,), jnp.int32),
            pltpu.VMEM((ROWS_PER_W + LANES,), jnp.int32),
            pltpu.VMEM((NUM_RELATIONS // 2, 2 * EMBED_DIM), jnp.float32),
            pltpu.VMEM((CHUNK, EMBED_DIM), jnp.float32),
            pltpu.VMEM((CHUNK, EMBED_DIM), jnp.float32),
            pltpu.SemaphoreType.DMA,
        ],
    )(heads, rels, ent3, rel2)


def kernel(heads, relations, entity_table, relation_table):
    heads = jnp.asarray(heads, jnp.int32)
    relations = jnp.asarray(relations, jnp.int32)
    rel2 = relation_table.reshape(NUM_RELATIONS // 2, 2 * EMBED_DIM)
    return _run(heads, relations, entity_table, rel2)


# speedup vs baseline: 12.7003x; 1.0147x over previous
import jax
import jax.numpy as jnp
from jax import lax
from jax.experimental import pallas as pl
from jax.experimental.pallas import tpu as pltpu
from jax.experimental.pallas import tpu_sc as plsc

NUM_ENTITIES = 1000000
NUM_RELATIONS = 1000
EMBED_DIM = 64
BATCH = 16384

NC = 2
NS = 16
NW = NC * NS
ROWS_PER_W = BATCH // NW
CHUNK = 64
NCHUNK = ROWS_PER_W // CHUNK
LANES = 16
NVEC = EMBED_DIM // LANES
GRP = 8
NGRP = NUM_ENTITIES // GRP

_GATHER_DNUMS = lax.GatherDimensionNumbers(
    offset_dims=(), collapsed_slice_dims=(0,), start_index_map=(0,))


def _permute(x, idx):
    return lax.gather(x, idx[:, None], _GATHER_DNUMS, (1,),
                      mode=lax.GatherScatterMode.PROMISE_IN_BOUNDS)


def _tec_body(heads_hbm, rels_hbm, ent_hbm, rel_hbm, out_hbm,
              hvec, rvec, reltab, ebuf0, ebuf1, outbuf, sem0, sem1):
    wid = lax.axis_index("c") * NS + lax.axis_index("s")
    base = wid * ROWS_PER_W

    pltpu.sync_copy(heads_hbm.at[pl.ds(base, ROWS_PER_W)],
                    hvec.at[pl.ds(0, ROWS_PER_W)])
    pltpu.sync_copy(rels_hbm.at[pl.ds(base, ROWS_PER_W)],
                    rvec.at[pl.ds(0, ROWS_PER_W)])
    pltpu.sync_copy(rel_hbm, reltab)

    iota = lax.iota(jnp.int32, LANES)
    perms = [iota ^ sh for sh in (8, 4, 2, 1)]

    def fire(c, ebuf, sem):
        def one(i, _):
            h = hvec[pl.ds(c * CHUNK + i, LANES)][0]
            pltpu.async_copy(
                ent_hbm.at[pl.ds(h, 1)], ebuf.at[pl.ds(i, 1)], sem)
            return 0

        lax.fori_loop(0, CHUNK, one, 0)

    def drain(ebuf, sem):
        pltpu.make_async_copy(out_hbm.at[pl.ds(0, CHUNK)], ebuf, sem).wait()

    def compute(c, ebuf):
        def row(i, _):
            r = rvec[pl.ds(c * CHUNK + i, LANES)][0]
            rq = r >> 1
            rp = (r & 1) * EMBED_DIM
            vs = []
            ss = None
            for k in range(NVEC):
                v = (ebuf[i, pl.ds(k * LANES, LANES)]
                     + reltab[rq, pl.ds(rp + k * LANES, LANES)])
                vs.append(v)
                sq = v * v
                ss = sq if ss is None else ss + sq
            for p in perms:
                ss = ss + _permute(ss, p)
            t = jnp.maximum(ss, jnp.float32(1e-24))
            bits = lax.bitcast_convert_type(t, jnp.int32)
            y = lax.bitcast_convert_type(
                jnp.int32(0x5F3759DF) - (bits >> 1), jnp.float32)
            for _ in range(3):
                y = y * (jnp.float32(1.5) - jnp.float32(0.5) * t * y * y)
            for k in range(NVEC):
                outbuf[i, pl.ds(k * LANES, LANES)] = vs[k] * y
            return 0

        lax.fori_loop(0, CHUNK, row, 0)
        pltpu.sync_copy(outbuf, out_hbm.at[pl.ds(base + c * CHUNK, CHUNK)])

    fire(0, ebuf0, sem0)

    def pair(k, _):
        c0 = 2 * k
        fire(c0 + 1, ebuf1, sem1)
        drain(ebuf0, sem0)
        compute(c0, ebuf0)

        @pl.when(k + 1 < NCHUNK // 2)
        def _():
            fire(c0 + 2, ebuf0, sem0)

        drain(ebuf1, sem1)
        compute(c0 + 1, ebuf1)
        return 0

    lax.fori_loop(0, NCHUNK // 2, pair, 0)


@jax.jit
def _run(heads, rels, ent3, rel2):
    mesh = plsc.VectorSubcoreMesh(
        core_axis_name="c", subcore_axis_name="s",
        num_cores=NC, num_subcores=NS)
    return pl.kernel(
        _tec_body,
        out_type=jax.ShapeDtypeStruct((BATCH, EMBED_DIM), jnp.float32),
        mesh=mesh,
        scratch_types=[
            pltpu.VMEM((ROWS_PER_W + LANES,), jnp.int32),
            pltpu.VMEM((ROWS_PER_W + LANES,), jnp.int32),
            pltpu.VMEM((NUM_RELATIONS // 2, 2 * EMBED_DIM), jnp.float32),
            pltpu.VMEM((CHUNK, EMBED_DIM), jnp.float32),
            pltpu.VMEM((CHUNK, EMBED_DIM), jnp.float32),
            pltpu.VMEM((CHUNK, EMBED_DIM), jnp.float32),
            pltpu.SemaphoreType.DMA,
            pltpu.SemaphoreType.DMA,
        ],
    )(heads, rels, ent3, rel2)


def kernel(heads, relations, entity_table, relation_table):
    heads = jnp.asarray(heads, jnp.int32)
    relations = jnp.asarray(relations, jnp.int32)
    rel2 = relation_table.reshape(NUM_RELATIONS // 2, 2 * EMBED_DIM)
    return _run(heads, relations, entity_table, rel2)
